# baseline (speedup 1.0000x reference)
import functools

import jax
import jax.numpy as jnp
from jax import lax
from jax.experimental import pallas as pl
from jax.experimental.pallas import tpu as pltpu
from jax.experimental.pallas import tpu_sc as plsc

B = 1024
S = 96
P = 336
K = 100000
TK = 16
DFF = 128
P2 = 384

KPAD = 106496
C2 = 8192
CLS = 2048
NSUB = C2 // CLS
NCHUNK = KPAD // C2
R2B = 256
R1B = 256
R3B = 128

NEGF = -3.0e38
IMAX = 2147483647
_INV_SQRT2 = 0.7071067811865476


def _gelu(x):
    return 0.5 * x * (1.0 + jax.lax.erf(x * _INV_SQRT2))


def _pre_body(x_ref, pred_W, pred_b, enc_W1, enc_b1, enc_W2, enc_b2,
              og_W1, og_b1, og_W2a, og_b2a, og_W2b, og_b2b, cg_W1a, cg_b1,
              y1_ref, feats_ref, scale_ref, shift_ref, cgn_ref):
    x = x_ref[...]
    mean = jnp.mean(x, axis=1, keepdims=True)
    xc = x - mean
    std = jnp.sqrt(jnp.sum(xc * xc, axis=1, keepdims=True) / (S - 1))
    std = jnp.where(std == 0.0, 1e-6, std)
    norm = xc / std
    y1_ref[...] = jnp.dot(x, pred_W[...], preferred_element_type=jnp.float32) + pred_b[...]
    h = jax.nn.relu(jnp.dot(norm, enc_W1[...], preferred_element_type=jnp.float32) + enc_b1[...])
    f = jnp.dot(h, enc_W2[...], preferred_element_type=jnp.float32) + enc_b2[...]
    fn = jnp.sqrt(jnp.sum(f * f, axis=1, keepdims=True))
    feats_ref[...] = f / jnp.maximum(fn, 1e-12)
    g = _gelu(jnp.dot(norm, og_W1[...], preferred_element_type=jnp.float32) + og_b1[...])
    scale_ref[...] = jnp.dot(g, og_W2a[...], preferred_element_type=jnp.float32) + og_b2a[...]
    shift_ref[...] = jnp.dot(g, og_W2b[...], preferred_element_type=jnp.float32) + og_b2b[...]
    cgn_ref[...] = jnp.dot(norm, cg_W1a[...], preferred_element_type=jnp.float32) + cg_b1[...]


def _merge2(av, ai, bv, bi):
    g = av >= bv
    return (jnp.where(g, av, bv), jnp.where(g, ai, bi),
            jnp.where(g, bv, av), jnp.where(g, bi, ai))


def _merge_pairs(a1v, a1i, a2v, a2i, b1v, b1i, b2v, b2i):
    g = a1v >= b1v
    c1v = jnp.where(g, a1v, b1v)
    c1i = jnp.where(g, a1i, b1i)
    xg = a2v >= b1v
    x_v = jnp.where(xg, a2v, b1v)
    x_i = jnp.where(xg, a2i, b1i)
    yg = b2v > a1v
    y_v = jnp.where(yg, b2v, a1v)
    y_i = jnp.where(yg, b2i, a1i)
    return c1v, c1i, jnp.where(g, x_v, y_v), jnp.where(g, x_i, y_i)


def _ret_body(feats_ref, pfT_ref, sim_ref, idx_ref, m1, i1, m2, i2):
    j = pl.program_id(1)

    @pl.when(j == 0)
    def _init():
        m1[...] = jnp.full((R2B, CLS), NEGF, jnp.float32)
        m2[...] = jnp.full((R2B, CLS), NEGF, jnp.float32)
        i1[...] = jnp.zeros((R2B, CLS), jnp.int32)
        i2[...] = jnp.zeros((R2B, CLS), jnp.int32)

    s = jnp.dot(feats_ref[...], pfT_ref[...], preferred_element_type=jnp.float32)
    iota = jax.lax.broadcasted_iota(jnp.int32, (R2B, CLS), 1)
    base = j * C2
    subs = []
    for q in range(NSUB):
        sq = s[:, q * CLS:(q + 1) * CLS]
        cq = base + q * CLS + iota
        subs.append((jnp.where(cq < K, sq, -1e30), cq))
    p0 = _merge2(*subs[0], *subs[1])
    p1 = _merge2(*subs[2], *subs[3])
    c = _merge_pairs(*p0, *p1)
    n1v, n1i, n2v, n2i = _merge_pairs(m1[...], i1[...], m2[...], i2[...], *c)
    m1[...] = n1v
    i1[...] = n1i
    m2[...] = n2v
    i2[...] = n2i

    @pl.when(j == NCHUNK - 1)
    def _peel():
        kiota = jax.lax.broadcasted_iota(jnp.int32, (R2B, TK), 1)

        def step(k, carry):
            v1, ii1, v2, ii2, simacc, idxacc = carry
            mv = jnp.max(v1, axis=1, keepdims=True)
            sel = jnp.min(jnp.where(v1 == mv, ii1, IMAX), axis=1, keepdims=True)
            simacc = jnp.where(kiota == k, mv, simacc)
            idxacc = jnp.where(kiota == k, sel, idxacc)
            hit = (v1 == mv) & (ii1 == sel)
            v1 = jnp.where(hit, v2, v1)
            ii1 = jnp.where(hit, ii2, ii1)
            v2 = jnp.where(hit, NEGF, v2)
            return v1, ii1, v2, ii2, simacc, idxacc

        init = (m1[...], i1[...], m2[...], i2[...],
                jnp.zeros((R2B, TK), jnp.float32), jnp.zeros((R2B, TK), jnp.int32))
        _, _, _, _, simacc, idxacc = jax.lax.fori_loop(0, TK, step, init)
        sim_ref[...] = simacc
        idx_ref[...] = idxacc


def _post_body(x_ref, sim_ref, tf_ref, y1_ref, scale_ref, shift_ref, cgn_ref,
               cg_W1f, cg_W2r, cg_b2, wm_W1, wm_b1, wm_W2, wm_b2,
               out_ref):
    x = x_ref[...]
    mean = jnp.mean(x, axis=1, keepdims=True)
    xc = x - mean
    std = jnp.sqrt(jnp.sum(xc * xc, axis=1, keepdims=True) / (S - 1))
    std = jnp.where(std == 0.0, 1e-6, std)

    cgn = cgn_ref[...]
    w1f = cg_W1f[...]
    w2r = cg_W2r[...]
    confs = []
    for k in range(TK):
        tfk = tf_ref[:, k, :]
        hk = _gelu(cgn + jnp.dot(tfk, w1f, preferred_element_type=jnp.float32))
        confs.append(jnp.sum(hk * w2r, axis=1, keepdims=True))
    conf = jnp.concatenate(confs, axis=1) + cg_b2[...]
    ts = sim_ref[...] * jax.nn.sigmoid(conf)

    m = jnp.max(ts, axis=1, keepdims=True)
    e = jnp.exp(ts - m)
    w = e / jnp.sum(e, axis=1, keepdims=True)

    fusion = jnp.zeros((x.shape[0], P2), jnp.float32)
    for k in range(TK):
        fusion = fusion + w[:, k:k + 1] * tf_ref[:, k, :]

    wh = _gelu(jnp.dot(ts, wm_W1[...], preferred_element_type=jnp.float32) + wm_b1[...])
    wl = jnp.dot(wh, wm_W2[...], preferred_element_type=jnp.float32) + wm_b2[...]
    wm = jnp.max(wl, axis=1, keepdims=True)
    we = jnp.exp(wl - wm)
    wsm = we / jnp.sum(we, axis=1, keepdims=True)

    seasonal = fusion * scale_ref[...] + shift_ref[...]
    y = seasonal * std + mean
    out_ref[...] = y * wsm[:, 0:1] + y1_ref[...] * wsm[:, 1:2]


def _copy_body(in_ref, out_ref):
    out_ref[...] = in_ref[...]


def _full(shape):
    return pl.BlockSpec(shape, lambda *args: tuple(0 for _ in shape))


def _sc_gather(table, idx_flat):
    info = plsc.get_sparse_core_info()
    nc, ns = info.num_cores, info.num_subcores
    nw = nc * ns
    n_idx = idx_flat.shape[0]
    b_per_w = n_idx // nw
    chunk = 128
    mesh = plsc.VectorSubcoreMesh(core_axis_name="c", subcore_axis_name="s")

    @functools.partial(
        pl.kernel, mesh=mesh,
        out_type=jax.ShapeDtypeStruct((n_idx, P2), jnp.float32),
        compiler_params=pltpu.CompilerParams(use_tc_tiling_on_sc=True),
        scratch_types=[
            pltpu.VMEM((chunk,), jnp.int32),
            pltpu.VMEM((chunk, P2), jnp.float32),
            pltpu.SemaphoreType.DMA,
        ],
    )
    def k(table_hbm, idx_hbm, out_hbm, idx_v, rows_v, sem):
        wid = lax.axis_index("s") * nc + lax.axis_index("c")
        for h in range(b_per_w // chunk):
            base = wid * b_per_w + h * chunk
            pltpu.sync_copy(idx_hbm.at[pl.ds(base, chunk)], idx_v)
            pltpu.async_copy(table_hbm.at[idx_v], rows_v, sem).wait()
            pltpu.sync_copy(rows_v, out_hbm.at[pl.ds(base, chunk)])

    return k(table, idx_flat)


def kernel(x_enc, x_mark_enc, x_dec, x_mark_dec, enc_W1, enc_b1, enc_W2, enc_b2,
           past_features, past_series, future_series,
           cg_W1, cg_b1, cg_W2, cg_b2, og_W1, og_b1, og_W2, og_b2,
           wm_W1, wm_b1, wm_W2, wm_b2, pred_W, pred_b):
    x = x_enc[..., 0]
    padP = ((0, 0), (0, P2 - P))
    og_W2a = jnp.pad(og_W2[:, :P], padP)
    og_W2b = jnp.pad(og_W2[:, P:], padP)
    og_b2a = jnp.pad(og_b2[:P], (0, P2 - P))
    og_b2b = jnp.pad(og_b2[P:], (0, P2 - P))
    cg_W1a = cg_W1[:S, :]
    cg_W1f = jnp.pad(cg_W1[S:, :], ((0, P2 - P), (0, 0)))
    pred_Wp = jnp.pad(pred_W, padP)
    pred_bp = jnp.pad(pred_b, (0, P2 - P))
    future_p = jnp.pad(future_series, padP)

    pre = pl.pallas_call(
        _pre_body,
        grid=(B // R1B,),
        in_specs=[
            pl.BlockSpec((R1B, S), lambda i: (i, 0)),
            _full((S, P2)), _full((1, P2)),
            _full((S, 64)), _full((1, 64)),
            _full((64, TK)), _full((1, TK)),
            _full((S, 2 * S)), _full((1, 2 * S)),
            _full((2 * S, P2)), _full((1, P2)),
            _full((2 * S, P2)), _full((1, P2)),
            _full((S, DFF)), _full((1, DFF)),
        ],
        out_specs=[
            pl.BlockSpec((R1B, P2), lambda i: (i, 0)),
            pl.BlockSpec((R1B, TK), lambda i: (i, 0)),
            pl.BlockSpec((R1B, P2), lambda i: (i, 0)),
            pl.BlockSpec((R1B, P2), lambda i: (i, 0)),
            pl.BlockSpec((R1B, DFF), lambda i: (i, 0)),
        ],
        out_shape=[
            jax.ShapeDtypeStruct((B, P2), jnp.float32),
            jax.ShapeDtypeStruct((B, TK), jnp.float32),
            jax.ShapeDtypeStruct((B, P2), jnp.float32),
            jax.ShapeDtypeStruct((B, P2), jnp.float32),
            jax.ShapeDtypeStruct((B, DFF), jnp.float32),
        ],
    )
    y1, feats, scale, shift, cgn = pre(
        x, pred_Wp, pred_bp[None, :], enc_W1, enc_b1[None, :], enc_W2, enc_b2[None, :],
        og_W1, og_b1[None, :], og_W2a, og_b2a[None, :], og_W2b, og_b2b[None, :],
        cg_W1a, cg_b1[None, :])

    pfT = jnp.pad(past_features.T, ((0, 0), (0, KPAD - K)))

    topk_sim, topk_idx = pl.pallas_call(
        _ret_body,
        grid=(B // R2B, NCHUNK),
        in_specs=[
            pl.BlockSpec((R2B, TK), lambda i, j: (i, 0)),
            pl.BlockSpec((TK, C2), lambda i, j: (0, j)),
        ],
        out_specs=[
            pl.BlockSpec((R2B, TK), lambda i, j: (i, 0)),
            pl.BlockSpec((R2B, TK), lambda i, j: (i, 0)),
        ],
        out_shape=[
            jax.ShapeDtypeStruct((B, TK), jnp.float32),
            jax.ShapeDtypeStruct((B, TK), jnp.int32),
        ],
        scratch_shapes=[
            pltpu.VMEM((R2B, CLS), jnp.float32),
            pltpu.VMEM((R2B, CLS), jnp.int32),
            pltpu.VMEM((R2B, CLS), jnp.float32),
            pltpu.VMEM((R2B, CLS), jnp.int32),
        ],
    )(feats, pfT)

    fs_std = pl.pallas_call(
        _copy_body,
        grid=(20,),
        in_specs=[pl.BlockSpec((K // 20, P), lambda i: (i, 0))],
        out_specs=pl.BlockSpec((K // 20, P), lambda i: (i, 0)),
        out_shape=jax.ShapeDtypeStruct((K, P), jnp.float32),
    )(future_series)
    topk_future = jnp.pad(jnp.take(fs_std, topk_idx, axis=0),
                          ((0, 0), (0, 0), (0, P2 - P)))

    out = pl.pallas_call(
        _post_body,
        grid=(B // R3B,),
        in_specs=[
            pl.BlockSpec((R3B, S), lambda i: (i, 0)),
            pl.BlockSpec((R3B, TK), lambda i: (i, 0)),
            pl.BlockSpec((R3B, TK, P2), lambda i: (i, 0, 0)),
            pl.BlockSpec((R3B, P2), lambda i: (i, 0)),
            pl.BlockSpec((R3B, P2), lambda i: (i, 0)),
            pl.BlockSpec((R3B, P2), lambda i: (i, 0)),
            pl.BlockSpec((R3B, DFF), lambda i: (i, 0)),
            _full((P2, DFF)), _full((1, DFF)), _full((1, TK)),
            _full((TK, DFF)), _full((1, DFF)),
            _full((DFF, 2)), _full((1, 2)),
        ],
        out_specs=pl.BlockSpec((R3B, P2), lambda i: (i, 0)),
        out_shape=jax.ShapeDtypeStruct((B, P2), jnp.float32),
    )(x, topk_sim, topk_future, y1, scale, shift, cgn,
      cg_W1f, cg_W2.T, jnp.broadcast_to(cg_b2[None, :], (1, TK)),
      wm_W1, wm_b1[None, :], wm_W2, wm_b2[None, :])

    return out[:, :P, None]


# speedup vs baseline: 1.3111x; 1.3111x over previous
import functools

import jax
import jax.numpy as jnp
from jax import lax
from jax.experimental import pallas as pl
from jax.experimental.pallas import tpu as pltpu
from jax.experimental.pallas import tpu_sc as plsc

B = 1024
S = 96
P = 336
K = 100000
TK = 16
DFF = 128
P2 = 384

KPAD = 106496
C2 = 8192
CLS = 2048
NSUB = C2 // CLS
NCHUNK = KPAD // C2
R2B = 256
R1B = 256
R3B = 128

NEGF = -3.0e38
IMAX = 2147483647
_INV_SQRT2 = 0.7071067811865476


def _gelu(x):
    return 0.5 * x * (1.0 + jax.lax.erf(x * _INV_SQRT2))


def _pre_body(x_ref, pred_W, pred_b, enc_W1, enc_b1, enc_W2, enc_b2,
              og_W1, og_b1, og_W2a, og_b2a, og_W2b, og_b2b, cg_W1a, cg_b1,
              y1_ref, feats_ref, scale_ref, shift_ref, cgn_ref):
    x = x_ref[...]
    mean = jnp.mean(x, axis=1, keepdims=True)
    xc = x - mean
    std = jnp.sqrt(jnp.sum(xc * xc, axis=1, keepdims=True) / (S - 1))
    std = jnp.where(std == 0.0, 1e-6, std)
    norm = xc / std
    y1_ref[...] = jnp.dot(x, pred_W[...], preferred_element_type=jnp.float32) + pred_b[...]
    h = jax.nn.relu(jnp.dot(norm, enc_W1[...], preferred_element_type=jnp.float32) + enc_b1[...])
    f = jnp.dot(h, enc_W2[...], preferred_element_type=jnp.float32) + enc_b2[...]
    fn = jnp.sqrt(jnp.sum(f * f, axis=1, keepdims=True))
    feats_ref[...] = f / jnp.maximum(fn, 1e-12)
    g = _gelu(jnp.dot(norm, og_W1[...], preferred_element_type=jnp.float32) + og_b1[...])
    scale_ref[...] = jnp.dot(g, og_W2a[...], preferred_element_type=jnp.float32) + og_b2a[...]
    shift_ref[...] = jnp.dot(g, og_W2b[...], preferred_element_type=jnp.float32) + og_b2b[...]
    cgn_ref[...] = jnp.dot(norm, cg_W1a[...], preferred_element_type=jnp.float32) + cg_b1[...]


def _merge2(av, ai, bv, bi):
    g = av >= bv
    return (jnp.where(g, av, bv), jnp.where(g, ai, bi),
            jnp.where(g, bv, av), jnp.where(g, bi, ai))


def _merge_pairs(a1v, a1i, a2v, a2i, b1v, b1i, b2v, b2i):
    g = a1v >= b1v
    c1v = jnp.where(g, a1v, b1v)
    c1i = jnp.where(g, a1i, b1i)
    xg = a2v >= b1v
    x_v = jnp.where(xg, a2v, b1v)
    x_i = jnp.where(xg, a2i, b1i)
    yg = b2v > a1v
    y_v = jnp.where(yg, b2v, a1v)
    y_i = jnp.where(yg, b2i, a1i)
    return c1v, c1i, jnp.where(g, x_v, y_v), jnp.where(g, x_i, y_i)


def _ret_body(feats_ref, pfT_ref, sim_ref, idx_ref, m1, i1, m2, i2):
    j = pl.program_id(1)

    @pl.when(j == 0)
    def _init():
        m1[...] = jnp.full((R2B, CLS), NEGF, jnp.float32)
        m2[...] = jnp.full((R2B, CLS), NEGF, jnp.float32)
        i1[...] = jnp.zeros((R2B, CLS), jnp.int32)
        i2[...] = jnp.zeros((R2B, CLS), jnp.int32)

    s = jnp.dot(feats_ref[...], pfT_ref[...], preferred_element_type=jnp.float32)
    iota = jax.lax.broadcasted_iota(jnp.int32, (R2B, CLS), 1)
    base = j * C2
    subs = []
    for q in range(NSUB):
        sq = s[:, q * CLS:(q + 1) * CLS]
        cq = base + q * CLS + iota
        subs.append((jnp.where(cq < K, sq, -1e30), cq))
    p0 = _merge2(*subs[0], *subs[1])
    p1 = _merge2(*subs[2], *subs[3])
    c = _merge_pairs(*p0, *p1)
    n1v, n1i, n2v, n2i = _merge_pairs(m1[...], i1[...], m2[...], i2[...], *c)
    m1[...] = n1v
    i1[...] = n1i
    m2[...] = n2v
    i2[...] = n2i

    @pl.when(j == NCHUNK - 1)
    def _peel():
        kiota = jax.lax.broadcasted_iota(jnp.int32, (R2B, TK), 1)

        def step(k, carry):
            v1, ii1, v2, ii2, simacc, idxacc = carry
            mv = jnp.max(v1, axis=1, keepdims=True)
            sel = jnp.min(jnp.where(v1 == mv, ii1, IMAX), axis=1, keepdims=True)
            simacc = jnp.where(kiota == k, mv, simacc)
            idxacc = jnp.where(kiota == k, sel, idxacc)
            hit = (v1 == mv) & (ii1 == sel)
            v1 = jnp.where(hit, v2, v1)
            ii1 = jnp.where(hit, ii2, ii1)
            v2 = jnp.where(hit, NEGF, v2)
            return v1, ii1, v2, ii2, simacc, idxacc

        init = (m1[...], i1[...], m2[...], i2[...],
                jnp.zeros((R2B, TK), jnp.float32), jnp.zeros((R2B, TK), jnp.int32))
        _, _, _, _, simacc, idxacc = jax.lax.fori_loop(0, TK, step, init)
        sim_ref[...] = simacc
        idx_ref[...] = idxacc


def _post_body(x_ref, sim_ref, tf_ref, y1_ref, scale_ref, shift_ref, cgn_ref,
               cg_W1f, cg_W2r, cg_b2, wm_W1, wm_b1, wm_W2, wm_b2,
               out_ref):
    x = x_ref[...]
    mean = jnp.mean(x, axis=1, keepdims=True)
    xc = x - mean
    std = jnp.sqrt(jnp.sum(xc * xc, axis=1, keepdims=True) / (S - 1))
    std = jnp.where(std == 0.0, 1e-6, std)

    cgn = cgn_ref[...]
    w1f = cg_W1f[...]
    w2r = cg_W2r[...]
    confs = []
    for k in range(TK):
        tfk = tf_ref[:, k, :]
        hk = _gelu(cgn + jnp.dot(tfk, w1f, preferred_element_type=jnp.float32))
        confs.append(jnp.sum(hk * w2r, axis=1, keepdims=True))
    conf = jnp.concatenate(confs, axis=1) + cg_b2[...]
    ts = sim_ref[...] * jax.nn.sigmoid(conf)

    m = jnp.max(ts, axis=1, keepdims=True)
    e = jnp.exp(ts - m)
    w = e / jnp.sum(e, axis=1, keepdims=True)

    fusion = jnp.zeros((x.shape[0], P2), jnp.float32)
    for k in range(TK):
        fusion = fusion + w[:, k:k + 1] * tf_ref[:, k, :]

    wh = _gelu(jnp.dot(ts, wm_W1[...], preferred_element_type=jnp.float32) + wm_b1[...])
    wl = jnp.dot(wh, wm_W2[...], preferred_element_type=jnp.float32) + wm_b2[...]
    wm = jnp.max(wl, axis=1, keepdims=True)
    we = jnp.exp(wl - wm)
    wsm = we / jnp.sum(we, axis=1, keepdims=True)

    seasonal = fusion * scale_ref[...] + shift_ref[...]
    y = seasonal * std + mean
    out_ref[...] = y * wsm[:, 0:1] + y1_ref[...] * wsm[:, 1:2]


def _copy_body(in_ref, out_ref):
    out_ref[...] = in_ref[...]


def _full(shape):
    return pl.BlockSpec(shape, lambda *args: tuple(0 for _ in shape))


def _sc_gather(table, idx_flat):
    info = plsc.get_sparse_core_info()
    nc, ns = info.num_cores, info.num_subcores
    nw = nc * ns
    n_idx = idx_flat.shape[0]
    b_per_w = n_idx // nw
    chunk = 128
    mesh = plsc.VectorSubcoreMesh(core_axis_name="c", subcore_axis_name="s")

    @functools.partial(
        pl.kernel, mesh=mesh,
        out_type=jax.ShapeDtypeStruct((n_idx, P2), jnp.float32),
        compiler_params=pltpu.CompilerParams(use_tc_tiling_on_sc=True),
        scratch_types=[
            pltpu.VMEM((chunk,), jnp.int32),
            pltpu.VMEM((chunk, P2), jnp.float32),
            pltpu.SemaphoreType.DMA,
        ],
    )
    def k(table_hbm, idx_hbm, out_hbm, idx_v, rows_v, sem):
        wid = lax.axis_index("s") * nc + lax.axis_index("c")
        for h in range(b_per_w // chunk):
            base = wid * b_per_w + h * chunk
            pltpu.sync_copy(idx_hbm.at[pl.ds(base, chunk)], idx_v)
            pltpu.async_copy(table_hbm.at[idx_v], rows_v, sem).wait()
            pltpu.sync_copy(rows_v, out_hbm.at[pl.ds(base, chunk)])

    return k(table, idx_flat)


def kernel(x_enc, x_mark_enc, x_dec, x_mark_dec, enc_W1, enc_b1, enc_W2, enc_b2,
           past_features, past_series, future_series,
           cg_W1, cg_b1, cg_W2, cg_b2, og_W1, og_b1, og_W2, og_b2,
           wm_W1, wm_b1, wm_W2, wm_b2, pred_W, pred_b):
    x = x_enc[..., 0]
    padP = ((0, 0), (0, P2 - P))
    og_W2a = jnp.pad(og_W2[:, :P], padP)
    og_W2b = jnp.pad(og_W2[:, P:], padP)
    og_b2a = jnp.pad(og_b2[:P], (0, P2 - P))
    og_b2b = jnp.pad(og_b2[P:], (0, P2 - P))
    cg_W1a = cg_W1[:S, :]
    cg_W1f = jnp.pad(cg_W1[S:, :], ((0, P2 - P), (0, 0)))
    pred_Wp = jnp.pad(pred_W, padP)
    pred_bp = jnp.pad(pred_b, (0, P2 - P))
    future_p = jnp.pad(future_series, padP)

    pre = pl.pallas_call(
        _pre_body,
        grid=(B // R1B,),
        in_specs=[
            pl.BlockSpec((R1B, S), lambda i: (i, 0)),
            _full((S, P2)), _full((1, P2)),
            _full((S, 64)), _full((1, 64)),
            _full((64, TK)), _full((1, TK)),
            _full((S, 2 * S)), _full((1, 2 * S)),
            _full((2 * S, P2)), _full((1, P2)),
            _full((2 * S, P2)), _full((1, P2)),
            _full((S, DFF)), _full((1, DFF)),
        ],
        out_specs=[
            pl.BlockSpec((R1B, P2), lambda i: (i, 0)),
            pl.BlockSpec((R1B, TK), lambda i: (i, 0)),
            pl.BlockSpec((R1B, P2), lambda i: (i, 0)),
            pl.BlockSpec((R1B, P2), lambda i: (i, 0)),
            pl.BlockSpec((R1B, DFF), lambda i: (i, 0)),
        ],
        out_shape=[
            jax.ShapeDtypeStruct((B, P2), jnp.float32),
            jax.ShapeDtypeStruct((B, TK), jnp.float32),
            jax.ShapeDtypeStruct((B, P2), jnp.float32),
            jax.ShapeDtypeStruct((B, P2), jnp.float32),
            jax.ShapeDtypeStruct((B, DFF), jnp.float32),
        ],
    )
    y1, feats, scale, shift, cgn = pre(
        x, pred_Wp, pred_bp[None, :], enc_W1, enc_b1[None, :], enc_W2, enc_b2[None, :],
        og_W1, og_b1[None, :], og_W2a, og_b2a[None, :], og_W2b, og_b2b[None, :],
        cg_W1a, cg_b1[None, :])

    pfT = jnp.pad(past_features.T, ((0, 0), (0, KPAD - K)))

    topk_sim, topk_idx = pl.pallas_call(
        _ret_body,
        grid=(B // R2B, NCHUNK),
        in_specs=[
            pl.BlockSpec((R2B, TK), lambda i, j: (i, 0)),
            pl.BlockSpec((TK, C2), lambda i, j: (0, j)),
        ],
        out_specs=[
            pl.BlockSpec((R2B, TK), lambda i, j: (i, 0)),
            pl.BlockSpec((R2B, TK), lambda i, j: (i, 0)),
        ],
        out_shape=[
            jax.ShapeDtypeStruct((B, TK), jnp.float32),
            jax.ShapeDtypeStruct((B, TK), jnp.int32),
        ],
        scratch_shapes=[
            pltpu.VMEM((R2B, CLS), jnp.float32),
            pltpu.VMEM((R2B, CLS), jnp.int32),
            pltpu.VMEM((R2B, CLS), jnp.float32),
            pltpu.VMEM((R2B, CLS), jnp.int32),
        ],
    )(feats, pfT)

    topk_future = jnp.pad(
        jnp.take(future_series.astype(jnp.bfloat16), topk_idx, axis=0)
        .astype(jnp.float32),
        ((0, 0), (0, 0), (0, P2 - P)))

    out = pl.pallas_call(
        _post_body,
        grid=(B // R3B,),
        in_specs=[
            pl.BlockSpec((R3B, S), lambda i: (i, 0)),
            pl.BlockSpec((R3B, TK), lambda i: (i, 0)),
            pl.BlockSpec((R3B, TK, P2), lambda i: (i, 0, 0)),
            pl.BlockSpec((R3B, P2), lambda i: (i, 0)),
            pl.BlockSpec((R3B, P2), lambda i: (i, 0)),
            pl.BlockSpec((R3B, P2), lambda i: (i, 0)),
            pl.BlockSpec((R3B, DFF), lambda i: (i, 0)),
            _full((P2, DFF)), _full((1, DFF)), _full((1, TK)),
            _full((TK, DFF)), _full((1, DFF)),
            _full((DFF, 2)), _full((1, 2)),
        ],
        out_specs=pl.BlockSpec((R3B, P2), lambda i: (i, 0)),
        out_shape=jax.ShapeDtypeStruct((B, P2), jnp.float32),
    )(x, topk_sim, topk_future, y1, scale, shift, cgn,
      cg_W1f, cg_W2.T, jnp.broadcast_to(cg_b2[None, :], (1, TK)),
      wm_W1, wm_b1[None, :], wm_W2, wm_b2[None, :])

    return out[:, :P, None]
